# baseline (speedup 1.0000x reference)
import functools

import jax
import jax.numpy as jnp
from jax import lax
from jax.experimental import pallas as pl
from jax.experimental.pallas import tpu as pltpu, tpu_sc as plsc

N = 10000
NP = 10240
D = 128
E = 160000
NC = 2
NS = 16
NW = NC * NS
EPW = E // NW
CH = 128
NCH = 40
EPWP = NCH * CH
RPT = NP // NS
BLK = 1280
GRID = NP // BLK
BN_EPS = 1e-5

_mesh = plsc.VectorSubcoreMesh(core_axis_name="c", subcore_axis_name="s")
_sc_params = pltpu.CompilerParams(needs_layout_passes=False)


def _fill(vref, val, n16):
    def body(i, carry):
        vref[pl.ds(i * 16, 16)] = jnp.full((16,), val, jnp.float32)
        return carry
    lax.fori_loop(0, n16, body, 0)


@functools.partial(
    pl.kernel,
    out_type=jax.ShapeDtypeStruct((NC, NP), jnp.float32),
    mesh=_mesh,
    compiler_params=_sc_params,
    scratch_types=[
        pltpu.VMEM((NCH, CH), jnp.int32),
        pltpu.VMEM((CH,), jnp.float32),
        pltpu.VMEM((RPT,), jnp.float32),
        pltpu.SemaphoreType.DMA,
        pltpu.VMEM_SHARED((NP,), jnp.float32),
    ],
)
def _sc_count(dstw, cnt_out, idx_v, ones_v, zeros_v, sem, cnt_sh):
    cid = lax.axis_index("c")
    sid = lax.axis_index("s")
    w = cid * NS + sid
    ld = pltpu.async_copy(dstw.at[w], idx_v, sem)
    _fill(ones_v, 1.0, CH // 16)
    _fill(zeros_v, 0.0, RPT // 16)
    pltpu.sync_copy(zeros_v, cnt_sh.at[pl.ds(sid * RPT, RPT)])
    ld.wait()
    plsc.subcore_barrier()

    def chunk(k, carry):
        pltpu.async_copy(ones_v, cnt_sh.at[idx_v.at[k]], sem, add=True)
        return carry
    lax.fori_loop(0, NCH, chunk, 0)

    def drain(k, carry):
        pltpu.make_async_copy(ones_v, cnt_sh.at[idx_v.at[0]], sem).wait()
        return carry
    lax.fori_loop(0, NCH, drain, 0)
    plsc.subcore_barrier()
    pltpu.sync_copy(cnt_sh.at[pl.ds(sid * RPT, RPT)],
                    cnt_out.at[cid, pl.ds(sid * RPT, RPT)])


@functools.partial(
    pl.kernel,
    out_type=jax.ShapeDtypeStruct((NC, NP, D), jnp.float32),
    mesh=_mesh,
    compiler_params=_sc_params,
    scratch_types=[
        pltpu.VMEM((NCH, CH), jnp.int32),
        pltpu.VMEM((NCH, CH), jnp.int32),
        pltpu.VMEM((CH, D), jnp.float32),
        pltpu.VMEM((CH, D), jnp.float32),
        pltpu.SemaphoreType.DMA,
        pltpu.SemaphoreType.DMA,
        pltpu.SemaphoreType.DMA,
        pltpu.SemaphoreType.DMA,
        pltpu.SemaphoreType.DMA,
        pltpu.VMEM_SHARED((NP, D), jnp.float32),
    ],
)
def _sc_segsum(g, srcw, dstw, acc_out, si_v, di_v, gbuf0, gbuf1,
               sem0, sem1, ssem0, ssem1, semz, acc_sh):
    cid = lax.axis_index("c")
    sid = lax.axis_index("s")
    w = cid * NS + sid
    ld_s = pltpu.async_copy(srcw.at[w], si_v, sem0)
    ld_d = pltpu.async_copy(dstw.at[w], di_v, sem1)

    def zb(i, carry):
        for j in range(D // 16):
            gbuf0[i, pl.ds(j * 16, 16)] = jnp.zeros((16,), jnp.float32)
        return carry
    lax.fori_loop(0, CH, zb, 0)
    for r in range(RPT // CH):
        pltpu.async_copy(gbuf0, acc_sh.at[pl.ds(sid * RPT + r * CH, CH)], semz)
    for r in range(RPT // CH):
        pltpu.make_async_copy(gbuf0, acc_sh.at[pl.ds(sid * RPT, CH)],
                              semz).wait()
    ld_s.wait()
    ld_d.wait()
    plsc.subcore_barrier()

    pltpu.async_copy(g.at[si_v.at[0]], gbuf0, sem0)
    pltpu.async_copy(g.at[si_v.at[1]], gbuf1, sem1)

    def pipe(t, carry):
        k0 = 2 * t
        pltpu.make_async_copy(g.at[si_v.at[k0]], gbuf0, sem0).wait()
        pltpu.async_copy(gbuf0, acc_sh.at[di_v.at[k0]], ssem0, add=True)
        pltpu.make_async_copy(g.at[si_v.at[k0 + 1]], gbuf1, sem1).wait()
        pltpu.async_copy(gbuf1, acc_sh.at[di_v.at[k0 + 1]], ssem1, add=True)

        @pl.when(t < NCH // 2 - 1)
        def _():
            pltpu.make_async_copy(gbuf0, acc_sh.at[di_v.at[0]], ssem0).wait()
            pltpu.async_copy(g.at[si_v.at[k0 + 2]], gbuf0, sem0)
            pltpu.make_async_copy(gbuf1, acc_sh.at[di_v.at[0]], ssem1).wait()
            pltpu.async_copy(g.at[si_v.at[k0 + 3]], gbuf1, sem1)
        return carry
    lax.fori_loop(0, NCH // 2, pipe, 0)
    pltpu.make_async_copy(gbuf0, acc_sh.at[di_v.at[0]], ssem0).wait()
    pltpu.make_async_copy(gbuf1, acc_sh.at[di_v.at[0]], ssem1).wait()
    plsc.subcore_barrier()
    for r in range(RPT // CH):
        off = sid * RPT + r * CH
        pltpu.async_copy(acc_sh.at[pl.ds(off, CH)],
                         acc_out.at[cid, pl.ds(off, CH)], semz)
    for r in range(RPT // CH):
        pltpu.make_async_copy(acc_sh.at[pl.ds(sid * RPT, CH)],
                              acc_out.at[cid, pl.ds(sid * RPT, CH)],
                              semz).wait()


@functools.partial(
    pl.kernel,
    out_type=jax.ShapeDtypeStruct((NC, NP), jnp.float32),
    mesh=_mesh,
    compiler_params=_sc_params,
    scratch_types=[
        pltpu.VMEM((NCH, CH), jnp.int32),
        pltpu.VMEM((NCH, CH), jnp.int32),
        pltpu.VMEM((NP,), jnp.float32),
        pltpu.VMEM((NP,), jnp.float32),
        pltpu.VMEM((NP,), jnp.float32),
        pltpu.VMEM((NCH, CH), jnp.float32),
        pltpu.VMEM((RPT,), jnp.float32),
        pltpu.SemaphoreType.DMA,
        pltpu.SemaphoreType.DMA,
        pltpu.VMEM_SHARED((NP,), jnp.float32),
    ],
)
def _sc_line1(a, b, dinv, srcw, dstw, m_out,
              si_v, di_v, a_v, b_v, w_v, val_v, zeros_v, sem, ssem, m_sh):
    cid = lax.axis_index("c")
    sid = lax.axis_index("s")
    w = cid * NS + sid
    cps = [pltpu.async_copy(srcw.at[w], si_v, sem),
           pltpu.async_copy(dstw.at[w], di_v, sem),
           pltpu.async_copy(a, a_v, sem),
           pltpu.async_copy(b, b_v, sem),
           pltpu.async_copy(dinv, w_v, sem)]
    _fill(zeros_v, 0.0, RPT // 16)
    pltpu.sync_copy(zeros_v, m_sh.at[pl.ds(sid * RPT, RPT)])
    for cp in cps:
        cp.wait()
    plsc.subcore_barrier()

    def chunk(k, carry):
        for j in range(CH // 16):
            s = si_v[k, pl.ds(j * 16, 16)]
            dd = di_v[k, pl.ds(j * 16, 16)]
            av = plsc.load_gather(a_v, [s])
            bv = plsc.load_gather(b_v, [dd])
            wv = plsc.load_gather(w_v, [s])
            val_v[k, pl.ds(j * 16, 16)] = wv * (av + bv)
        pltpu.async_copy(val_v.at[k], m_sh.at[di_v.at[k]], ssem, add=True)
        return carry
    lax.fori_loop(0, NCH, chunk, 0)

    def drain(k, carry):
        pltpu.make_async_copy(val_v.at[0], m_sh.at[di_v.at[0]], ssem).wait()
        return carry
    lax.fori_loop(0, NCH, drain, 0)
    plsc.subcore_barrier()
    pltpu.sync_copy(m_sh.at[pl.ds(sid * RPT, RPT)],
                    m_out.at[cid, pl.ds(sid * RPT, RPT)])


@functools.partial(
    pl.kernel,
    out_type=jax.ShapeDtypeStruct((E,), jnp.float32),
    mesh=_mesh,
    compiler_params=_sc_params,
    scratch_types=[
        pltpu.VMEM((NCH, CH), jnp.int32),
        pltpu.VMEM((NCH, CH), jnp.int32),
        pltpu.VMEM((NP,), jnp.float32),
        pltpu.VMEM((NP,), jnp.float32),
        pltpu.VMEM((NP,), jnp.float32),
        pltpu.VMEM((NP,), jnp.float32),
        pltpu.VMEM((NP,), jnp.float32),
        pltpu.VMEM((16,), jnp.float32),
        pltpu.VMEM((EPWP,), jnp.float32),
        pltpu.SemaphoreType.DMA,
    ],
)
def _sc_line2(a, b, dinv, mparts, b3, srcw, dstw, out,
              si_v, di_v, a_v, b_v, w_v, m_v, m2_v, b3_v, o_v, sem):
    cid = lax.axis_index("c")
    sid = lax.axis_index("s")
    w = cid * NS + sid
    cps = [pltpu.async_copy(srcw.at[w], si_v, sem),
           pltpu.async_copy(dstw.at[w], di_v, sem),
           pltpu.async_copy(a, a_v, sem),
           pltpu.async_copy(b, b_v, sem),
           pltpu.async_copy(dinv, w_v, sem),
           pltpu.async_copy(mparts.at[0], m_v, sem),
           pltpu.async_copy(mparts.at[1], m2_v, sem),
           pltpu.async_copy(b3, b3_v, sem)]
    for cp in cps:
        cp.wait()

    def madd(i, carry):
        off = pl.ds(i * 16, 16)
        m_v[off] = m_v[off] + m2_v[off]
        return carry
    lax.fori_loop(0, NP // 16, madd, 0)
    b3v = b3_v[...]

    def chunk(k, carry):
        for j in range(CH // 16):
            s = si_v[k, pl.ds(j * 16, 16)]
            dd = di_v[k, pl.ds(j * 16, 16)]
            av = plsc.load_gather(a_v, [s])
            bv = plsc.load_gather(b_v, [dd])
            wv = plsc.load_gather(w_v, [s])
            mv = plsc.load_gather(m_v, [s])
            z = wv * mv + wv * wv * (av + bv) + b3v
            o_v[pl.ds(k * CH + j * 16, 16)] = 1.0 / (1.0 + jnp.exp(-z))
        return carry
    lax.fori_loop(0, NCH, chunk, 0)
    pltpu.sync_copy(o_v.at[pl.ds(0, EPW)], out.at[pl.ds(w * EPW, EPW)])


def _tc_prep1_body(x_ref, w1_ref, cnt_ref, g1_ref, dinv_ref):
    cnt = cnt_ref[0] + cnt_ref[1]
    dinv = lax.rsqrt(cnt + 1.0)
    h = jnp.dot(x_ref[...], w1_ref[...], preferred_element_type=jnp.float32)
    g1_ref[...] = dinv * h
    dinv_ref[...] = dinv


def _tc_prep1(xp, W1, cnt3):
    return pl.pallas_call(
        _tc_prep1_body,
        grid=(GRID,),
        in_specs=[
            pl.BlockSpec((BLK, D), lambda j: (j, 0)),
            pl.BlockSpec((D, D), lambda j: (0, 0)),
            pl.BlockSpec((NC, BLK, 1), lambda j: (0, j, 0)),
        ],
        out_specs=[
            pl.BlockSpec((BLK, D), lambda j: (j, 0)),
            pl.BlockSpec((BLK, 1), lambda j: (j, 0)),
        ],
        out_shape=[
            jax.ShapeDtypeStruct((NP, D), jnp.float32),
            jax.ShapeDtypeStruct((NP, 1), jnp.float32),
        ],
    )(xp, W1, cnt3)


def _tc_out1_body(acc_ref, g1_ref, dinv_ref, b1_ref, o_ref, st_ref):
    j = pl.program_id(0)
    accs = acc_ref[0] + acc_ref[1] + g1_ref[...]
    o = dinv_ref[...] * accs + b1_ref[...]
    o_ref[...] = o
    rows = j * BLK + lax.broadcasted_iota(jnp.int32, (BLK, 1), 0)
    om = jnp.where(rows < N, o, 0.0)
    st_ref[0, 0, :] = jnp.sum(om, axis=0)
    st_ref[0, 1, :] = jnp.sum(om * o, axis=0)


def _tc_out1(acc1, g1, dinv, b1r):
    return pl.pallas_call(
        _tc_out1_body,
        grid=(GRID,),
        in_specs=[
            pl.BlockSpec((NC, BLK, D), lambda j: (0, j, 0)),
            pl.BlockSpec((BLK, D), lambda j: (j, 0)),
            pl.BlockSpec((BLK, 1), lambda j: (j, 0)),
            pl.BlockSpec((1, D), lambda j: (0, 0)),
        ],
        out_specs=[
            pl.BlockSpec((BLK, D), lambda j: (j, 0)),
            pl.BlockSpec((1, 2, D), lambda j: (j, 0, 0)),
        ],
        out_shape=[
            jax.ShapeDtypeStruct((NP, D), jnp.float32),
            jax.ShapeDtypeStruct((GRID, 2, D), jnp.float32),
        ],
    )(acc1, g1, dinv, b1r)


def _tc_bn_l2_body(o1_ref, st_ref, dinv_ref, w2_ref, b2_ref, gam_ref, bet_ref,
                   g2_ref):
    j = pl.program_id(0)
    sums = jnp.sum(st_ref[...], axis=0)
    mean = sums[0:1] / N
    var = sums[1:2] / N - mean * mean
    inv = lax.rsqrt(var + BN_EPS)
    o = gam_ref[...] * (o1_ref[...] - mean) * inv + bet_ref[...]
    o = jnp.maximum(o, 0.0)
    h2 = jnp.dot(o, w2_ref[...], preferred_element_type=jnp.float32)
    rows = j * BLK + lax.broadcasted_iota(jnp.int32, (BLK, 1), 0)
    g2_ref[...] = jnp.where(rows < N, dinv_ref[...] * h2, 0.0)


def _tc_bn_l2(o1, stats, dinv, W2, b2r, gamr, betr):
    return pl.pallas_call(
        _tc_bn_l2_body,
        grid=(GRID,),
        in_specs=[
            pl.BlockSpec((BLK, D), lambda j: (j, 0)),
            pl.BlockSpec((GRID, 2, D), lambda j: (0, 0, 0)),
            pl.BlockSpec((BLK, 1), lambda j: (j, 0)),
            pl.BlockSpec((D, D), lambda j: (0, 0)),
            pl.BlockSpec((1, D), lambda j: (0, 0)),
            pl.BlockSpec((1, D), lambda j: (0, 0)),
            pl.BlockSpec((1, D), lambda j: (0, 0)),
        ],
        out_specs=pl.BlockSpec((BLK, D), lambda j: (j, 0)),
        out_shape=jax.ShapeDtypeStruct((NP, D), jnp.float32),
    )(o1, stats, dinv, W2, b2r, gamr, betr)


def _tc_fin_body(acc_ref, g2_ref, dinv_ref, b2_ref, w3a_ref, w3b_ref,
                 a_ref, b_ref):
    accs = acc_ref[0] + acc_ref[1] + g2_ref[...]
    o2 = jnp.maximum(dinv_ref[...] * accs + b2_ref[...], 0.0)
    a_ref[...] = jnp.dot(o2, w3a_ref[...], preferred_element_type=jnp.float32)
    b_ref[...] = jnp.dot(o2, w3b_ref[...], preferred_element_type=jnp.float32)


def _tc_fin(acc2, g2, dinv, b2r, w3a, w3b):
    return pl.pallas_call(
        _tc_fin_body,
        grid=(GRID,),
        in_specs=[
            pl.BlockSpec((NC, BLK, D), lambda j: (0, j, 0)),
            pl.BlockSpec((BLK, D), lambda j: (j, 0)),
            pl.BlockSpec((BLK, 1), lambda j: (j, 0)),
            pl.BlockSpec((1, D), lambda j: (0, 0)),
            pl.BlockSpec((D, 1), lambda j: (0, 0)),
            pl.BlockSpec((D, 1), lambda j: (0, 0)),
        ],
        out_specs=[
            pl.BlockSpec((BLK, 1), lambda j: (j, 0)),
            pl.BlockSpec((BLK, 1), lambda j: (j, 0)),
        ],
        out_shape=[
            jax.ShapeDtypeStruct((NP, 1), jnp.float32),
            jax.ShapeDtypeStruct((NP, 1), jnp.float32),
        ],
    )(acc2, g2, dinv, b2r, w3a, w3b)


def kernel(x, edge_index, W1, b1, gamma1, beta1, W2, b2, W3, b3):
    src = edge_index[0].astype(jnp.int32)
    dst = edge_index[1].astype(jnp.int32)
    npad = EPWP - EPW
    padv = N + (jnp.arange(npad, dtype=jnp.int32) % (NP - N))
    padw = jnp.broadcast_to(padv, (NW, npad))
    srcw = jnp.concatenate([src.reshape(NW, EPW), padw], axis=1)
    dstw = jnp.concatenate([dst.reshape(NW, EPW), padw], axis=1)
    srcw = srcw.reshape(NW, NCH, CH)
    dstw = dstw.reshape(NW, NCH, CH)

    xp = jnp.pad(x, ((0, NP - N), (0, 0)))
    b1r = b1.reshape(1, D)
    b2r = b2.reshape(1, D)
    gamr = gamma1.reshape(1, D)
    betr = beta1.reshape(1, D)
    w3a = W3[:D]
    w3b = W3[D:]
    b3r = jnp.full((16,), b3[0], jnp.float32)

    cnt2 = _sc_count(dstw)
    cnt3 = cnt2.reshape(NC, NP, 1)
    g1, dinv = _tc_prep1(xp, W1, cnt3)
    acc1 = _sc_segsum(g1, srcw, dstw)
    o1, stats = _tc_out1(acc1, g1, dinv, b1r)
    g2 = _tc_bn_l2(o1, stats, dinv, W2, b2r, gamr, betr)
    acc2 = _sc_segsum(g2, srcw, dstw)
    av, bv = _tc_fin(acc2, g2, dinv, b2r, w3a, w3b)
    af = av.reshape(NP)
    bf = bv.reshape(NP)
    dinvf = dinv.reshape(NP)
    mparts = _sc_line1(af, bf, dinvf, srcw, dstw)
    oute = _sc_line2(af, bf, dinvf, mparts, b3r, srcw, dstw)
    return oute.reshape(E, 1)


# speedup vs baseline: 1.1490x; 1.1490x over previous
import functools

import jax
import jax.numpy as jnp
from jax import lax
from jax.experimental import pallas as pl
from jax.experimental.pallas import tpu as pltpu, tpu_sc as plsc

N = 10000
NP = 10240
D = 128
E = 160000
NC = 2
NS = 16
NW = NC * NS
EPW = E // NW
CH = 128
NCH = 40
EPWP = NCH * CH
RPT = NP // NS
BLK = 1280
GRID = NP // BLK
BN_EPS = 1e-5

_mesh = plsc.VectorSubcoreMesh(core_axis_name="c", subcore_axis_name="s")
_sc_params = pltpu.CompilerParams(needs_layout_passes=False)


def _fill(vref, val, n16):
    def body(i, carry):
        vref[pl.ds(i * 16, 16)] = jnp.full((16,), val, jnp.float32)
        return carry
    lax.fori_loop(0, n16, body, 0)


@functools.partial(
    pl.kernel,
    out_type=jax.ShapeDtypeStruct((NC, NP), jnp.float32),
    mesh=_mesh,
    compiler_params=_sc_params,
    scratch_types=[
        pltpu.VMEM((NCH, CH), jnp.int32),
        pltpu.VMEM((CH,), jnp.float32),
        pltpu.VMEM((RPT,), jnp.float32),
        pltpu.SemaphoreType.DMA,
        pltpu.VMEM_SHARED((NP,), jnp.float32),
    ],
)
def _sc_count(dstw, cnt_out, idx_v, ones_v, zeros_v, sem, cnt_sh):
    cid = lax.axis_index("c")
    sid = lax.axis_index("s")
    w = cid * NS + sid
    ld = pltpu.async_copy(dstw.at[w], idx_v, sem)
    _fill(ones_v, 1.0, CH // 16)
    _fill(zeros_v, 0.0, RPT // 16)
    pltpu.sync_copy(zeros_v, cnt_sh.at[pl.ds(sid * RPT, RPT)])
    ld.wait()
    plsc.subcore_barrier()

    def chunk(k, carry):
        pltpu.async_copy(ones_v, cnt_sh.at[idx_v.at[k]], sem, add=True)
        return carry
    lax.fori_loop(0, NCH, chunk, 0)

    def drain(k, carry):
        pltpu.make_async_copy(ones_v, cnt_sh.at[idx_v.at[0]], sem).wait()
        return carry
    lax.fori_loop(0, NCH, drain, 0)
    plsc.subcore_barrier()
    pltpu.sync_copy(cnt_sh.at[pl.ds(sid * RPT, RPT)],
                    cnt_out.at[cid, pl.ds(sid * RPT, RPT)])


@functools.partial(
    pl.kernel,
    out_type=jax.ShapeDtypeStruct((NC, NP, D), jnp.float32),
    mesh=_mesh,
    compiler_params=_sc_params,
    scratch_types=[
        pltpu.VMEM((NCH, CH), jnp.int32),
        pltpu.VMEM((NCH, CH), jnp.int32),
        pltpu.VMEM((CH, D), jnp.float32),
        pltpu.VMEM((CH, D), jnp.float32),
        pltpu.SemaphoreType.DMA,
        pltpu.SemaphoreType.DMA,
        pltpu.SemaphoreType.DMA,
        pltpu.SemaphoreType.DMA,
        pltpu.SemaphoreType.DMA,
        pltpu.VMEM_SHARED((NP, D), jnp.float32),
    ],
)
def _sc_segsum(g, srcw, dstw, acc_out, si_v, di_v, gbuf0, gbuf1,
               sem0, sem1, ssem0, ssem1, semz, acc_sh):
    cid = lax.axis_index("c")
    sid = lax.axis_index("s")
    w = cid * NS + sid
    ld_s = pltpu.async_copy(srcw.at[w], si_v, sem0)
    ld_d = pltpu.async_copy(dstw.at[w], di_v, sem1)

    def zb(i, carry):
        for j in range(D // 16):
            gbuf0[i, pl.ds(j * 16, 16)] = jnp.zeros((16,), jnp.float32)
        return carry
    lax.fori_loop(0, CH, zb, 0)
    for r in range(RPT // CH):
        pltpu.async_copy(gbuf0, acc_sh.at[pl.ds(sid * RPT + r * CH, CH)], semz)
    for r in range(RPT // CH):
        pltpu.make_async_copy(gbuf0, acc_sh.at[pl.ds(sid * RPT, CH)],
                              semz).wait()
    ld_s.wait()
    ld_d.wait()
    plsc.subcore_barrier()

    pltpu.async_copy(g.at[si_v.at[0]], gbuf0, sem0)

    def pipe(t, carry):
        k0 = 2 * t
        pltpu.async_copy(g.at[si_v.at[k0 + 1]], gbuf1, sem1)
        pltpu.make_async_copy(g.at[si_v.at[k0]], gbuf0, sem0).wait()
        pltpu.sync_copy(gbuf0, acc_sh.at[di_v.at[k0]], add=True)

        @pl.when(t < NCH // 2 - 1)
        def _():
            pltpu.async_copy(g.at[si_v.at[k0 + 2]], gbuf0, sem0)
        pltpu.make_async_copy(g.at[si_v.at[k0 + 1]], gbuf1, sem1).wait()
        pltpu.sync_copy(gbuf1, acc_sh.at[di_v.at[k0 + 1]], add=True)
        return carry
    lax.fori_loop(0, NCH // 2, pipe, 0)
    plsc.subcore_barrier()
    for r in range(RPT // CH):
        off = sid * RPT + r * CH
        pltpu.async_copy(acc_sh.at[pl.ds(off, CH)],
                         acc_out.at[cid, pl.ds(off, CH)], semz)
    for r in range(RPT // CH):
        pltpu.make_async_copy(acc_sh.at[pl.ds(sid * RPT, CH)],
                              acc_out.at[cid, pl.ds(sid * RPT, CH)],
                              semz).wait()


@functools.partial(
    pl.kernel,
    out_type=jax.ShapeDtypeStruct((NC, NP), jnp.float32),
    mesh=_mesh,
    compiler_params=_sc_params,
    scratch_types=[
        pltpu.VMEM((NCH, CH), jnp.int32),
        pltpu.VMEM((NCH, CH), jnp.int32),
        pltpu.VMEM((NP,), jnp.float32),
        pltpu.VMEM((NP,), jnp.float32),
        pltpu.VMEM((NP,), jnp.float32),
        pltpu.VMEM((NCH, CH), jnp.float32),
        pltpu.VMEM((RPT,), jnp.float32),
        pltpu.SemaphoreType.DMA,
        pltpu.SemaphoreType.DMA,
        pltpu.VMEM_SHARED((NP,), jnp.float32),
    ],
)
def _sc_line1(a, b, dinv, srcw, dstw, m_out,
              si_v, di_v, a_v, b_v, w_v, val_v, zeros_v, sem, ssem, m_sh):
    cid = lax.axis_index("c")
    sid = lax.axis_index("s")
    w = cid * NS + sid
    cps = [pltpu.async_copy(srcw.at[w], si_v, sem),
           pltpu.async_copy(dstw.at[w], di_v, sem),
           pltpu.async_copy(a, a_v, sem),
           pltpu.async_copy(b, b_v, sem),
           pltpu.async_copy(dinv, w_v, sem)]
    _fill(zeros_v, 0.0, RPT // 16)
    pltpu.sync_copy(zeros_v, m_sh.at[pl.ds(sid * RPT, RPT)])
    for cp in cps:
        cp.wait()
    plsc.subcore_barrier()

    def chunk(k, carry):
        for j in range(CH // 16):
            s = si_v[k, pl.ds(j * 16, 16)]
            dd = di_v[k, pl.ds(j * 16, 16)]
            av = plsc.load_gather(a_v, [s])
            bv = plsc.load_gather(b_v, [dd])
            wv = plsc.load_gather(w_v, [s])
            val_v[k, pl.ds(j * 16, 16)] = wv * (av + bv)
        pltpu.async_copy(val_v.at[k], m_sh.at[di_v.at[k]], ssem, add=True)
        return carry
    lax.fori_loop(0, NCH, chunk, 0)

    def drain(k, carry):
        pltpu.make_async_copy(val_v.at[0], m_sh.at[di_v.at[0]], ssem).wait()
        return carry
    lax.fori_loop(0, NCH, drain, 0)
    plsc.subcore_barrier()
    pltpu.sync_copy(m_sh.at[pl.ds(sid * RPT, RPT)],
                    m_out.at[cid, pl.ds(sid * RPT, RPT)])


@functools.partial(
    pl.kernel,
    out_type=jax.ShapeDtypeStruct((E,), jnp.float32),
    mesh=_mesh,
    compiler_params=_sc_params,
    scratch_types=[
        pltpu.VMEM((NCH, CH), jnp.int32),
        pltpu.VMEM((NCH, CH), jnp.int32),
        pltpu.VMEM((NP,), jnp.float32),
        pltpu.VMEM((NP,), jnp.float32),
        pltpu.VMEM((NP,), jnp.float32),
        pltpu.VMEM((NP,), jnp.float32),
        pltpu.VMEM((NP,), jnp.float32),
        pltpu.VMEM((16,), jnp.float32),
        pltpu.VMEM((EPWP,), jnp.float32),
        pltpu.SemaphoreType.DMA,
    ],
)
def _sc_line2(a, b, dinv, mparts, b3, srcw, dstw, out,
              si_v, di_v, a_v, b_v, w_v, m_v, m2_v, b3_v, o_v, sem):
    cid = lax.axis_index("c")
    sid = lax.axis_index("s")
    w = cid * NS + sid
    cps = [pltpu.async_copy(srcw.at[w], si_v, sem),
           pltpu.async_copy(dstw.at[w], di_v, sem),
           pltpu.async_copy(a, a_v, sem),
           pltpu.async_copy(b, b_v, sem),
           pltpu.async_copy(dinv, w_v, sem),
           pltpu.async_copy(mparts.at[0], m_v, sem),
           pltpu.async_copy(mparts.at[1], m2_v, sem),
           pltpu.async_copy(b3, b3_v, sem)]
    for cp in cps:
        cp.wait()

    def madd(i, carry):
        off = pl.ds(i * 16, 16)
        m_v[off] = m_v[off] + m2_v[off]
        return carry
    lax.fori_loop(0, NP // 16, madd, 0)
    b3v = b3_v[...]

    def chunk(k, carry):
        for j in range(CH // 16):
            s = si_v[k, pl.ds(j * 16, 16)]
            dd = di_v[k, pl.ds(j * 16, 16)]
            av = plsc.load_gather(a_v, [s])
            bv = plsc.load_gather(b_v, [dd])
            wv = plsc.load_gather(w_v, [s])
            mv = plsc.load_gather(m_v, [s])
            z = wv * mv + wv * wv * (av + bv) + b3v
            o_v[pl.ds(k * CH + j * 16, 16)] = 1.0 / (1.0 + jnp.exp(-z))
        return carry
    lax.fori_loop(0, NCH, chunk, 0)
    pltpu.sync_copy(o_v.at[pl.ds(0, EPW)], out.at[pl.ds(w * EPW, EPW)])


def _tc_prep1_body(x_ref, w1_ref, cnt_ref, g1_ref, dinv_ref):
    cnt = cnt_ref[0] + cnt_ref[1]
    dinv = lax.rsqrt(cnt + 1.0)
    h = jnp.dot(x_ref[...], w1_ref[...], preferred_element_type=jnp.float32)
    g1_ref[...] = dinv * h
    dinv_ref[...] = dinv


def _tc_prep1(xp, W1, cnt3):
    return pl.pallas_call(
        _tc_prep1_body,
        grid=(GRID,),
        in_specs=[
            pl.BlockSpec((BLK, D), lambda j: (j, 0)),
            pl.BlockSpec((D, D), lambda j: (0, 0)),
            pl.BlockSpec((NC, BLK, 1), lambda j: (0, j, 0)),
        ],
        out_specs=[
            pl.BlockSpec((BLK, D), lambda j: (j, 0)),
            pl.BlockSpec((BLK, 1), lambda j: (j, 0)),
        ],
        out_shape=[
            jax.ShapeDtypeStruct((NP, D), jnp.float32),
            jax.ShapeDtypeStruct((NP, 1), jnp.float32),
        ],
    )(xp, W1, cnt3)


def _tc_out1_body(acc_ref, g1_ref, dinv_ref, b1_ref, o_ref, st_ref):
    j = pl.program_id(0)
    accs = acc_ref[0] + acc_ref[1] + g1_ref[...]
    o = dinv_ref[...] * accs + b1_ref[...]
    o_ref[...] = o
    rows = j * BLK + lax.broadcasted_iota(jnp.int32, (BLK, 1), 0)
    om = jnp.where(rows < N, o, 0.0)
    st_ref[0, 0, :] = jnp.sum(om, axis=0)
    st_ref[0, 1, :] = jnp.sum(om * o, axis=0)


def _tc_out1(acc1, g1, dinv, b1r):
    return pl.pallas_call(
        _tc_out1_body,
        grid=(GRID,),
        in_specs=[
            pl.BlockSpec((NC, BLK, D), lambda j: (0, j, 0)),
            pl.BlockSpec((BLK, D), lambda j: (j, 0)),
            pl.BlockSpec((BLK, 1), lambda j: (j, 0)),
            pl.BlockSpec((1, D), lambda j: (0, 0)),
        ],
        out_specs=[
            pl.BlockSpec((BLK, D), lambda j: (j, 0)),
            pl.BlockSpec((1, 2, D), lambda j: (j, 0, 0)),
        ],
        out_shape=[
            jax.ShapeDtypeStruct((NP, D), jnp.float32),
            jax.ShapeDtypeStruct((GRID, 2, D), jnp.float32),
        ],
    )(acc1, g1, dinv, b1r)


def _tc_bn_l2_body(o1_ref, st_ref, dinv_ref, w2_ref, b2_ref, gam_ref, bet_ref,
                   g2_ref):
    j = pl.program_id(0)
    sums = jnp.sum(st_ref[...], axis=0)
    mean = sums[0:1] / N
    var = sums[1:2] / N - mean * mean
    inv = lax.rsqrt(var + BN_EPS)
    o = gam_ref[...] * (o1_ref[...] - mean) * inv + bet_ref[...]
    o = jnp.maximum(o, 0.0)
    h2 = jnp.dot(o, w2_ref[...], preferred_element_type=jnp.float32)
    rows = j * BLK + lax.broadcasted_iota(jnp.int32, (BLK, 1), 0)
    g2_ref[...] = jnp.where(rows < N, dinv_ref[...] * h2, 0.0)


def _tc_bn_l2(o1, stats, dinv, W2, b2r, gamr, betr):
    return pl.pallas_call(
        _tc_bn_l2_body,
        grid=(GRID,),
        in_specs=[
            pl.BlockSpec((BLK, D), lambda j: (j, 0)),
            pl.BlockSpec((GRID, 2, D), lambda j: (0, 0, 0)),
            pl.BlockSpec((BLK, 1), lambda j: (j, 0)),
            pl.BlockSpec((D, D), lambda j: (0, 0)),
            pl.BlockSpec((1, D), lambda j: (0, 0)),
            pl.BlockSpec((1, D), lambda j: (0, 0)),
            pl.BlockSpec((1, D), lambda j: (0, 0)),
        ],
        out_specs=pl.BlockSpec((BLK, D), lambda j: (j, 0)),
        out_shape=jax.ShapeDtypeStruct((NP, D), jnp.float32),
    )(o1, stats, dinv, W2, b2r, gamr, betr)


def _tc_fin_body(acc_ref, g2_ref, dinv_ref, b2_ref, w3a_ref, w3b_ref,
                 a_ref, b_ref):
    accs = acc_ref[0] + acc_ref[1] + g2_ref[...]
    o2 = jnp.maximum(dinv_ref[...] * accs + b2_ref[...], 0.0)
    a_ref[...] = jnp.dot(o2, w3a_ref[...], preferred_element_type=jnp.float32)
    b_ref[...] = jnp.dot(o2, w3b_ref[...], preferred_element_type=jnp.float32)


def _tc_fin(acc2, g2, dinv, b2r, w3a, w3b):
    return pl.pallas_call(
        _tc_fin_body,
        grid=(GRID,),
        in_specs=[
            pl.BlockSpec((NC, BLK, D), lambda j: (0, j, 0)),
            pl.BlockSpec((BLK, D), lambda j: (j, 0)),
            pl.BlockSpec((BLK, 1), lambda j: (j, 0)),
            pl.BlockSpec((1, D), lambda j: (0, 0)),
            pl.BlockSpec((D, 1), lambda j: (0, 0)),
            pl.BlockSpec((D, 1), lambda j: (0, 0)),
        ],
        out_specs=[
            pl.BlockSpec((BLK, 1), lambda j: (j, 0)),
            pl.BlockSpec((BLK, 1), lambda j: (j, 0)),
        ],
        out_shape=[
            jax.ShapeDtypeStruct((NP, 1), jnp.float32),
            jax.ShapeDtypeStruct((NP, 1), jnp.float32),
        ],
    )(acc2, g2, dinv, b2r, w3a, w3b)


def kernel(x, edge_index, W1, b1, gamma1, beta1, W2, b2, W3, b3):
    src = edge_index[0].astype(jnp.int32)
    dst = edge_index[1].astype(jnp.int32)
    npad = EPWP - EPW
    padv = N + (jnp.arange(npad, dtype=jnp.int32) % (NP - N))
    padw = jnp.broadcast_to(padv, (NW, npad))
    srcw = jnp.concatenate([src.reshape(NW, EPW), padw], axis=1)
    dstw = jnp.concatenate([dst.reshape(NW, EPW), padw], axis=1)
    srcw = srcw.reshape(NW, NCH, CH)
    dstw = dstw.reshape(NW, NCH, CH)

    xp = jnp.pad(x, ((0, NP - N), (0, 0)))
    b1r = b1.reshape(1, D)
    b2r = b2.reshape(1, D)
    gamr = gamma1.reshape(1, D)
    betr = beta1.reshape(1, D)
    w3a = W3[:D]
    w3b = W3[D:]
    b3r = jnp.full((16,), b3[0], jnp.float32)

    cnt2 = _sc_count(dstw)
    cnt3 = cnt2.reshape(NC, NP, 1)
    g1, dinv = _tc_prep1(xp, W1, cnt3)
    acc1 = _sc_segsum(g1, srcw, dstw)
    o1, stats = _tc_out1(acc1, g1, dinv, b1r)
    g2 = _tc_bn_l2(o1, stats, dinv, W2, b2r, gamr, betr)
    acc2 = _sc_segsum(g2, srcw, dstw)
    av, bv = _tc_fin(acc2, g2, dinv, b2r, w3a, w3b)
    af = av.reshape(NP)
    bf = bv.reshape(NP)
    dinvf = dinv.reshape(NP)
    mparts = _sc_line1(af, bf, dinvf, srcw, dstw)
    oute = _sc_line2(af, bf, dinvf, mparts, b3r, srcw, dstw)
    return oute.reshape(E, 1)


# speedup vs baseline: 1.1525x; 1.0031x over previous
import functools

import jax
import jax.numpy as jnp
from jax import lax
from jax.experimental import pallas as pl
from jax.experimental.pallas import tpu as pltpu, tpu_sc as plsc

N = 10000
NP = 10240
D = 128
E = 160000
NC = 2
NS = 16
NW = NC * NS
EPW = E // NW
CH = 128
NCH = 40
EPWP = NCH * CH
RPT = NP // NS
BLK = 1000
GRID = N // BLK
BN_EPS = 1e-5

_mesh = plsc.VectorSubcoreMesh(core_axis_name="c", subcore_axis_name="s")
_sc_params = pltpu.CompilerParams(needs_layout_passes=False)


def _fill(vref, val, n16):
    def body(i, carry):
        vref[pl.ds(i * 16, 16)] = jnp.full((16,), val, jnp.float32)
        return carry
    lax.fori_loop(0, n16, body, 0)


@functools.partial(
    pl.kernel,
    out_type=jax.ShapeDtypeStruct((NC, NP), jnp.float32),
    mesh=_mesh,
    compiler_params=_sc_params,
    scratch_types=[
        pltpu.VMEM((NCH, CH), jnp.int32),
        pltpu.VMEM((CH,), jnp.float32),
        pltpu.VMEM((RPT,), jnp.float32),
        pltpu.SemaphoreType.DMA,
        pltpu.VMEM_SHARED((NP,), jnp.float32),
    ],
)
def _sc_count(dstw, cnt_out, idx_v, ones_v, zeros_v, sem, cnt_sh):
    cid = lax.axis_index("c")
    sid = lax.axis_index("s")
    w = cid * NS + sid
    ld = pltpu.async_copy(dstw.at[w], idx_v, sem)
    _fill(ones_v, 1.0, CH // 16)
    _fill(zeros_v, 0.0, RPT // 16)
    pltpu.sync_copy(zeros_v, cnt_sh.at[pl.ds(sid * RPT, RPT)])
    ld.wait()
    plsc.subcore_barrier()

    def chunk(k, carry):
        pltpu.async_copy(ones_v, cnt_sh.at[idx_v.at[k]], sem, add=True)
        return carry
    lax.fori_loop(0, NCH, chunk, 0)

    def drain(k, carry):
        pltpu.make_async_copy(ones_v, cnt_sh.at[idx_v.at[0]], sem).wait()
        return carry
    lax.fori_loop(0, NCH, drain, 0)
    plsc.subcore_barrier()
    pltpu.sync_copy(cnt_sh.at[pl.ds(sid * RPT, RPT)],
                    cnt_out.at[cid, pl.ds(sid * RPT, RPT)])


@functools.partial(
    pl.kernel,
    out_type=jax.ShapeDtypeStruct((NC, NP, D), jnp.float32),
    mesh=_mesh,
    compiler_params=_sc_params,
    scratch_types=[
        pltpu.VMEM((NCH, CH), jnp.int32),
        pltpu.VMEM((NCH, CH), jnp.int32),
        pltpu.VMEM((CH, D), jnp.float32),
        pltpu.VMEM((CH, D), jnp.float32),
        pltpu.SemaphoreType.DMA,
        pltpu.SemaphoreType.DMA,
        pltpu.SemaphoreType.DMA,
        pltpu.SemaphoreType.DMA,
        pltpu.SemaphoreType.DMA,
        pltpu.VMEM_SHARED((NP, D), jnp.float32),
    ],
)
def _sc_segsum(g, srcw, dstw, acc_out, si_v, di_v, gbuf0, gbuf1,
               sem0, sem1, ssem0, ssem1, semz, acc_sh):
    cid = lax.axis_index("c")
    sid = lax.axis_index("s")
    w = cid * NS + sid
    ld_s = pltpu.async_copy(srcw.at[w], si_v, sem0)
    ld_d = pltpu.async_copy(dstw.at[w], di_v, sem1)

    def zb(i, carry):
        for j in range(D // 16):
            gbuf0[i, pl.ds(j * 16, 16)] = jnp.zeros((16,), jnp.float32)
        return carry
    lax.fori_loop(0, CH, zb, 0)
    for r in range(RPT // CH):
        pltpu.async_copy(gbuf0, acc_sh.at[pl.ds(sid * RPT + r * CH, CH)], semz)
    for r in range(RPT // CH):
        pltpu.make_async_copy(gbuf0, acc_sh.at[pl.ds(sid * RPT, CH)],
                              semz).wait()
    ld_s.wait()
    ld_d.wait()
    plsc.subcore_barrier()

    pltpu.async_copy(g.at[si_v.at[0]], gbuf0, sem0)

    def pipe(t, carry):
        k0 = 2 * t
        pltpu.async_copy(g.at[si_v.at[k0 + 1]], gbuf1, sem1)
        pltpu.make_async_copy(g.at[si_v.at[k0]], gbuf0, sem0).wait()
        pltpu.sync_copy(gbuf0, acc_sh.at[di_v.at[k0]], add=True)

        @pl.when(t < NCH // 2 - 1)
        def _():
            pltpu.async_copy(g.at[si_v.at[k0 + 2]], gbuf0, sem0)
        pltpu.make_async_copy(g.at[si_v.at[k0 + 1]], gbuf1, sem1).wait()
        pltpu.sync_copy(gbuf1, acc_sh.at[di_v.at[k0 + 1]], add=True)
        return carry
    lax.fori_loop(0, NCH // 2, pipe, 0)
    plsc.subcore_barrier()
    for r in range(RPT // CH):
        off = sid * RPT + r * CH
        pltpu.async_copy(acc_sh.at[pl.ds(off, CH)],
                         acc_out.at[cid, pl.ds(off, CH)], semz)
    for r in range(RPT // CH):
        pltpu.make_async_copy(acc_sh.at[pl.ds(sid * RPT, CH)],
                              acc_out.at[cid, pl.ds(sid * RPT, CH)],
                              semz).wait()


@functools.partial(
    pl.kernel,
    out_type=jax.ShapeDtypeStruct((NC, NP), jnp.float32),
    mesh=_mesh,
    compiler_params=_sc_params,
    scratch_types=[
        pltpu.VMEM((NCH, CH), jnp.int32),
        pltpu.VMEM((NCH, CH), jnp.int32),
        pltpu.VMEM((NP,), jnp.float32),
        pltpu.VMEM((NP,), jnp.float32),
        pltpu.VMEM((NP,), jnp.float32),
        pltpu.VMEM((NCH, CH), jnp.float32),
        pltpu.VMEM((RPT,), jnp.float32),
        pltpu.SemaphoreType.DMA,
        pltpu.SemaphoreType.DMA,
        pltpu.VMEM_SHARED((NP,), jnp.float32),
    ],
)
def _sc_line1(a, b, dinv, srcw, dstw, m_out,
              si_v, di_v, a_v, b_v, w_v, val_v, zeros_v, sem, ssem, m_sh):
    cid = lax.axis_index("c")
    sid = lax.axis_index("s")
    w = cid * NS + sid
    cps = [pltpu.async_copy(srcw.at[w], si_v, sem),
           pltpu.async_copy(dstw.at[w], di_v, sem),
           pltpu.async_copy(a, a_v.at[pl.ds(0, N)], sem),
           pltpu.async_copy(b, b_v.at[pl.ds(0, N)], sem),
           pltpu.async_copy(dinv, w_v.at[pl.ds(0, N)], sem)]
    _fill(zeros_v, 0.0, RPT // 16)
    pltpu.sync_copy(zeros_v, m_sh.at[pl.ds(sid * RPT, RPT)])
    def tail(i, carry):
        b_v[pl.ds(N + i * 16, 16)] = jnp.zeros((16,), jnp.float32)
        return carry
    lax.fori_loop(0, (NP - N) // 16, tail, 0)
    for cp in cps:
        cp.wait()
    plsc.subcore_barrier()

    def chunk(k, carry):
        for j in range(CH // 16):
            s = si_v[k, pl.ds(j * 16, 16)]
            dd = di_v[k, pl.ds(j * 16, 16)]
            av = plsc.load_gather(a_v, [s])
            bv = plsc.load_gather(b_v, [dd])
            wv = plsc.load_gather(w_v, [s])
            val_v[k, pl.ds(j * 16, 16)] = wv * (av + bv)
        pltpu.async_copy(val_v.at[k], m_sh.at[di_v.at[k]], ssem, add=True)
        return carry
    lax.fori_loop(0, NCH, chunk, 0)

    def drain(k, carry):
        pltpu.make_async_copy(val_v.at[0], m_sh.at[di_v.at[0]], ssem).wait()
        return carry
    lax.fori_loop(0, NCH, drain, 0)
    plsc.subcore_barrier()
    pltpu.sync_copy(m_sh.at[pl.ds(sid * RPT, RPT)],
                    m_out.at[cid, pl.ds(sid * RPT, RPT)])


@functools.partial(
    pl.kernel,
    out_type=jax.ShapeDtypeStruct((E,), jnp.float32),
    mesh=_mesh,
    compiler_params=_sc_params,
    scratch_types=[
        pltpu.VMEM((NCH, CH), jnp.int32),
        pltpu.VMEM((NCH, CH), jnp.int32),
        pltpu.VMEM((NP,), jnp.float32),
        pltpu.VMEM((NP,), jnp.float32),
        pltpu.VMEM((NP,), jnp.float32),
        pltpu.VMEM((NP,), jnp.float32),
        pltpu.VMEM((NP,), jnp.float32),
        pltpu.VMEM((16,), jnp.float32),
        pltpu.VMEM((EPWP,), jnp.float32),
        pltpu.SemaphoreType.DMA,
    ],
)
def _sc_line2(a, b, dinv, mparts, b3, srcw, dstw, out,
              si_v, di_v, a_v, b_v, w_v, m_v, m2_v, b3_v, o_v, sem):
    cid = lax.axis_index("c")
    sid = lax.axis_index("s")
    w = cid * NS + sid
    cps = [pltpu.async_copy(srcw.at[w], si_v, sem),
           pltpu.async_copy(dstw.at[w], di_v, sem),
           pltpu.async_copy(a, a_v.at[pl.ds(0, N)], sem),
           pltpu.async_copy(b, b_v.at[pl.ds(0, N)], sem),
           pltpu.async_copy(dinv, w_v.at[pl.ds(0, N)], sem),
           pltpu.async_copy(mparts.at[0], m_v, sem),
           pltpu.async_copy(mparts.at[1], m2_v, sem),
           pltpu.async_copy(b3, b3_v, sem)]
    def tail(i, carry):
        b_v[pl.ds(N + i * 16, 16)] = jnp.zeros((16,), jnp.float32)
        return carry
    lax.fori_loop(0, (NP - N) // 16, tail, 0)
    for cp in cps:
        cp.wait()

    def madd(i, carry):
        off = pl.ds(i * 16, 16)
        m_v[off] = m_v[off] + m2_v[off]
        return carry
    lax.fori_loop(0, NP // 16, madd, 0)
    b3v = b3_v[...]

    def chunk(k, carry):
        for j in range(CH // 16):
            s = si_v[k, pl.ds(j * 16, 16)]
            dd = di_v[k, pl.ds(j * 16, 16)]
            av = plsc.load_gather(a_v, [s])
            bv = plsc.load_gather(b_v, [dd])
            wv = plsc.load_gather(w_v, [s])
            mv = plsc.load_gather(m_v, [s])
            z = wv * mv + wv * wv * (av + bv) + b3v
            o_v[pl.ds(k * CH + j * 16, 16)] = 1.0 / (1.0 + jnp.exp(-z))
        return carry
    lax.fori_loop(0, NCH, chunk, 0)
    pltpu.sync_copy(o_v.at[pl.ds(0, EPW)], out.at[pl.ds(w * EPW, EPW)])


def _tc_prep1_body(x_ref, w1_ref, cnt_ref, g1_ref, dinv_ref):
    cnt = cnt_ref[0] + cnt_ref[1]
    dinv = lax.rsqrt(cnt + 1.0)
    h = jnp.dot(x_ref[...], w1_ref[...], preferred_element_type=jnp.float32)
    g1_ref[...] = dinv * h
    dinv_ref[...] = dinv


def _tc_prep1(x, W1, cnt31):
    return pl.pallas_call(
        _tc_prep1_body,
        grid=(GRID,),
        in_specs=[
            pl.BlockSpec((BLK, D), lambda j: (j, 0)),
            pl.BlockSpec((D, D), lambda j: (0, 0)),
            pl.BlockSpec((NC, BLK, 1), lambda j: (0, j, 0)),
        ],
        out_specs=[
            pl.BlockSpec((BLK, D), lambda j: (j, 0)),
            pl.BlockSpec((BLK, 1), lambda j: (j, 0)),
        ],
        out_shape=[
            jax.ShapeDtypeStruct((N, D), jnp.float32),
            jax.ShapeDtypeStruct((N, 1), jnp.float32),
        ],
    )(x, W1, cnt31)


def _tc_mid_body(acc_ref, g1_ref, dinv_ref, b1_ref, w2_ref, gam_ref,
                 bet_ref, g2_ref, o1_s, st_s):
    p = pl.program_id(0)
    j = pl.program_id(1)

    @pl.when(p == 0)
    def _():
        accs = acc_ref[0] + acc_ref[1] + g1_ref[...]
        o = dinv_ref[...] * accs + b1_ref[...]
        o1_s[pl.ds(j * BLK, BLK), :] = o
        s1 = jnp.sum(o, axis=0, keepdims=True)
        s2 = jnp.sum(o * o, axis=0, keepdims=True)
        st = jnp.concatenate([s1, s2], axis=0)

        @pl.when(j == 0)
        def _():
            st_s[...] = st

        @pl.when(j > 0)
        def _():
            st_s[...] = st_s[...] + st

    @pl.when(p == 1)
    def _():
        mean = st_s[0:1] / N
        var = st_s[1:2] / N - mean * mean
        inv = lax.rsqrt(var + BN_EPS)
        o1 = o1_s[pl.ds(j * BLK, BLK), :]
        o = gam_ref[...] * (o1 - mean) * inv + bet_ref[...]
        o = jnp.maximum(o, 0.0)
        h2 = jnp.dot(o, w2_ref[...], preferred_element_type=jnp.float32)
        g2_ref[...] = dinv_ref[...] * h2


def _tc_mid(acc1, g1, dinv, b1r, W2, gamr, betr):
    return pl.pallas_call(
        _tc_mid_body,
        grid=(2, GRID),
        in_specs=[
            pl.BlockSpec((NC, BLK, D), lambda p, j: (0, j * (1 - p), 0)),
            pl.BlockSpec((BLK, D), lambda p, j: (j * (1 - p), 0)),
            pl.BlockSpec((BLK, 1), lambda p, j: (j, 0)),
            pl.BlockSpec((1, D), lambda p, j: (0, 0)),
            pl.BlockSpec((D, D), lambda p, j: (0, 0)),
            pl.BlockSpec((1, D), lambda p, j: (0, 0)),
            pl.BlockSpec((1, D), lambda p, j: (0, 0)),
        ],
        out_specs=pl.BlockSpec((BLK, D), lambda p, j: (j, 0)),
        out_shape=jax.ShapeDtypeStruct((N, D), jnp.float32),
        scratch_shapes=[
            pltpu.VMEM((N, D), jnp.float32),
            pltpu.VMEM((2, D), jnp.float32),
        ],
    )(acc1, g1, dinv, b1r, W2, gamr, betr)


def _tc_fin_body(acc_ref, g2_ref, dinv_ref, b2_ref, w3a_ref, w3b_ref,
                 a_ref, b_ref):
    accs = acc_ref[0] + acc_ref[1] + g2_ref[...]
    o2 = jnp.maximum(dinv_ref[...] * accs + b2_ref[...], 0.0)
    a_ref[...] = jnp.dot(o2, w3a_ref[...], preferred_element_type=jnp.float32)
    b_ref[...] = jnp.dot(o2, w3b_ref[...], preferred_element_type=jnp.float32)


def _tc_fin(acc2, g2, dinv, b2r, w3a, w3b):
    return pl.pallas_call(
        _tc_fin_body,
        grid=(GRID,),
        in_specs=[
            pl.BlockSpec((NC, BLK, D), lambda j: (0, j, 0)),
            pl.BlockSpec((BLK, D), lambda j: (j, 0)),
            pl.BlockSpec((BLK, 1), lambda j: (j, 0)),
            pl.BlockSpec((1, D), lambda j: (0, 0)),
            pl.BlockSpec((D, 1), lambda j: (0, 0)),
            pl.BlockSpec((D, 1), lambda j: (0, 0)),
        ],
        out_specs=[
            pl.BlockSpec((BLK, 1), lambda j: (j, 0)),
            pl.BlockSpec((BLK, 1), lambda j: (j, 0)),
        ],
        out_shape=[
            jax.ShapeDtypeStruct((N, 1), jnp.float32),
            jax.ShapeDtypeStruct((N, 1), jnp.float32),
        ],
    )(acc2, g2, dinv, b2r, w3a, w3b)


def kernel(x, edge_index, W1, b1, gamma1, beta1, W2, b2, W3, b3):
    src = edge_index[0].astype(jnp.int32)
    dst = edge_index[1].astype(jnp.int32)
    npad = EPWP - EPW
    padv = jnp.arange(npad, dtype=jnp.int32) % (NP - N)
    pads = jnp.broadcast_to(padv, (NW, npad))
    padd = jnp.broadcast_to(N + padv, (NW, npad))
    srcw = jnp.concatenate([src.reshape(NW, EPW), pads], axis=1)
    dstw = jnp.concatenate([dst.reshape(NW, EPW), padd], axis=1)
    srcw = srcw.reshape(NW, NCH, CH)
    dstw = dstw.reshape(NW, NCH, CH)

    b1r = b1.reshape(1, D)
    b2r = b2.reshape(1, D)
    gamr = gamma1.reshape(1, D)
    betr = beta1.reshape(1, D)
    w3a = W3[:D]
    w3b = W3[D:]
    b3r = jnp.full((16,), b3[0], jnp.float32)

    cnt31 = _sc_count(dstw).reshape(NC, NP, 1)
    g1, dinv = _tc_prep1(x, W1, cnt31)
    acc1 = _sc_segsum(g1, srcw, dstw)
    g2 = _tc_mid(acc1, g1, dinv, b1r, W2, gamr, betr)
    acc2 = _sc_segsum(g2, srcw, dstw)
    av, bv = _tc_fin(acc2, g2, dinv, b2r, w3a, w3b)
    af = av.reshape(N)
    bf = bv.reshape(N)
    dinvf = dinv.reshape(N)
    mparts = _sc_line1(af, bf, dinvf, srcw, dstw)
    oute = _sc_line2(af, bf, dinvf, mparts, b3r, srcw, dstw)
    return oute.reshape(E, 1)


# speedup vs baseline: 1.1540x; 1.0013x over previous
import functools

import jax
import jax.numpy as jnp
from jax import lax
from jax.experimental import pallas as pl
from jax.experimental.pallas import tpu as pltpu, tpu_sc as plsc

N = 10000
NP = 10240
D = 128
E = 160000
NC = 2
NS = 16
NW = NC * NS
EPW = E // NW
CH = 128
NCH = 40
EPWP = NCH * CH
RPT = NP // NS
BLK = 1000
GRID = N // BLK
BN_EPS = 1e-5

_mesh = plsc.VectorSubcoreMesh(core_axis_name="c", subcore_axis_name="s")
_sc_params = pltpu.CompilerParams(needs_layout_passes=False)


def _fill(vref, val, n16):
    def body(i, carry):
        vref[pl.ds(i * 16, 16)] = jnp.full((16,), val, jnp.float32)
        return carry
    lax.fori_loop(0, n16, body, 0)


@functools.partial(
    pl.kernel,
    out_type=jax.ShapeDtypeStruct((NC, NP), jnp.float32),
    mesh=_mesh,
    compiler_params=_sc_params,
    scratch_types=[
        pltpu.VMEM((NCH, CH), jnp.int32),
        pltpu.VMEM((CH,), jnp.float32),
        pltpu.VMEM((RPT,), jnp.float32),
        pltpu.SemaphoreType.DMA,
        pltpu.VMEM_SHARED((NP,), jnp.float32),
    ],
)
def _sc_count(dstw, cnt_out, idx_v, ones_v, zeros_v, sem, cnt_sh):
    cid = lax.axis_index("c")
    sid = lax.axis_index("s")
    w = cid * NS + sid
    ld = pltpu.async_copy(dstw.at[w], idx_v, sem)
    _fill(ones_v, 1.0, CH // 16)
    _fill(zeros_v, 0.0, RPT // 16)
    pltpu.sync_copy(zeros_v, cnt_sh.at[pl.ds(sid * RPT, RPT)])
    ld.wait()
    plsc.subcore_barrier()

    def chunk(k, carry):
        pltpu.async_copy(ones_v, cnt_sh.at[idx_v.at[k]], sem, add=True)
        return carry
    lax.fori_loop(0, NCH, chunk, 0)

    def drain(k, carry):
        pltpu.make_async_copy(ones_v, cnt_sh.at[idx_v.at[0]], sem).wait()
        return carry
    lax.fori_loop(0, NCH, drain, 0)
    plsc.subcore_barrier()
    pltpu.sync_copy(cnt_sh.at[pl.ds(sid * RPT, RPT)],
                    cnt_out.at[cid, pl.ds(sid * RPT, RPT)])


@functools.partial(
    pl.kernel,
    out_type=jax.ShapeDtypeStruct((NC, NP, D), jnp.float32),
    mesh=_mesh,
    compiler_params=_sc_params,
    scratch_types=[
        pltpu.VMEM((NCH, CH), jnp.int32),
        pltpu.VMEM((NCH, CH), jnp.int32),
        pltpu.VMEM((CH, D), jnp.float32),
        pltpu.VMEM((CH, D), jnp.float32),
        pltpu.SemaphoreType.DMA,
        pltpu.SemaphoreType.DMA,
        pltpu.SemaphoreType.DMA,
        pltpu.SemaphoreType.DMA,
        pltpu.SemaphoreType.DMA,
        pltpu.VMEM_SHARED((NP, D), jnp.float32),
    ],
)
def _sc_segsum(g, srcw, dstw, acc_out, si_v, di_v, gbuf0, gbuf1,
               sem0, sem1, ssem0, ssem1, semz, acc_sh):
    cid = lax.axis_index("c")
    sid = lax.axis_index("s")
    w = cid * NS + sid
    ld_s = pltpu.async_copy(srcw.at[w], si_v, sem0)
    ld_d = pltpu.async_copy(dstw.at[w], di_v, sem1)

    def zb(i, carry):
        for j in range(D // 16):
            gbuf0[i, pl.ds(j * 16, 16)] = jnp.zeros((16,), jnp.float32)
        return carry
    lax.fori_loop(0, CH, zb, 0)
    for r in range(RPT // CH):
        pltpu.async_copy(gbuf0, acc_sh.at[pl.ds(sid * RPT + r * CH, CH)], semz)
    for r in range(RPT // CH):
        pltpu.make_async_copy(gbuf0, acc_sh.at[pl.ds(sid * RPT, CH)],
                              semz).wait()
    ld_s.wait()
    ld_d.wait()
    plsc.subcore_barrier()

    pltpu.async_copy(g.at[si_v.at[0]], gbuf0, sem0)

    def pipe(t, carry):
        k0 = 2 * t
        pltpu.async_copy(g.at[si_v.at[k0 + 1]], gbuf1, sem1)
        pltpu.make_async_copy(g.at[si_v.at[k0]], gbuf0, sem0).wait()
        pltpu.sync_copy(gbuf0, acc_sh.at[di_v.at[k0]], add=True)

        @pl.when(t < NCH // 2 - 1)
        def _():
            pltpu.async_copy(g.at[si_v.at[k0 + 2]], gbuf0, sem0)
        pltpu.make_async_copy(g.at[si_v.at[k0 + 1]], gbuf1, sem1).wait()
        pltpu.sync_copy(gbuf1, acc_sh.at[di_v.at[k0 + 1]], add=True)
        return carry
    lax.fori_loop(0, NCH // 2, pipe, 0)
    plsc.subcore_barrier()
    for r in range(RPT // CH):
        off = sid * RPT + r * CH
        pltpu.async_copy(acc_sh.at[pl.ds(off, CH)],
                         acc_out.at[cid, pl.ds(off, CH)], semz)
    for r in range(RPT // CH):
        pltpu.make_async_copy(acc_sh.at[pl.ds(sid * RPT, CH)],
                              acc_out.at[cid, pl.ds(sid * RPT, CH)],
                              semz).wait()


@functools.partial(
    pl.kernel,
    out_type=jax.ShapeDtypeStruct((NC, NP), jnp.float32),
    mesh=_mesh,
    compiler_params=_sc_params,
    scratch_types=[
        pltpu.VMEM((NCH, CH), jnp.int32),
        pltpu.VMEM((NCH, CH), jnp.int32),
        pltpu.VMEM((NP,), jnp.float32),
        pltpu.VMEM((NP,), jnp.float32),
        pltpu.VMEM((NP,), jnp.float32),
        pltpu.VMEM((NCH, CH), jnp.float32),
        pltpu.VMEM((RPT,), jnp.float32),
        pltpu.SemaphoreType.DMA,
        pltpu.SemaphoreType.DMA,
        pltpu.VMEM_SHARED((NP,), jnp.float32),
    ],
)
def _sc_line1(a, b, dinv, srcw, dstw, m_out,
              si_v, di_v, a_v, b_v, w_v, val_v, zeros_v, sem, ssem, m_sh):
    cid = lax.axis_index("c")
    sid = lax.axis_index("s")
    w = cid * NS + sid
    cps = [pltpu.async_copy(srcw.at[w], si_v, sem),
           pltpu.async_copy(dstw.at[w], di_v, sem),
           pltpu.async_copy(a, a_v.at[pl.ds(0, N)], sem),
           pltpu.async_copy(b, b_v.at[pl.ds(0, N)], sem),
           pltpu.async_copy(dinv, w_v.at[pl.ds(0, N)], sem)]
    _fill(zeros_v, 0.0, RPT // 16)
    pltpu.sync_copy(zeros_v, m_sh.at[pl.ds(sid * RPT, RPT)])
    def tail(i, carry):
        b_v[pl.ds(N + i * 16, 16)] = jnp.zeros((16,), jnp.float32)
        return carry
    lax.fori_loop(0, (NP - N) // 16, tail, 0)
    for cp in cps:
        cp.wait()
    plsc.subcore_barrier()

    def chunk(k, carry):
        for j in range(CH // 16):
            s = si_v[k, pl.ds(j * 16, 16)]
            dd = di_v[k, pl.ds(j * 16, 16)]
            av = plsc.load_gather(a_v, [s])
            bv = plsc.load_gather(b_v, [dd])
            wv = plsc.load_gather(w_v, [s])
            val_v[k, pl.ds(j * 16, 16)] = wv * (av + bv)
        pltpu.async_copy(val_v.at[k], m_sh.at[di_v.at[k]], ssem, add=True)
        return carry
    lax.fori_loop(0, NCH, chunk, 0)

    def drain(k, carry):
        pltpu.make_async_copy(val_v.at[0], m_sh.at[di_v.at[0]], ssem).wait()
        return carry
    lax.fori_loop(0, NCH, drain, 0)
    plsc.subcore_barrier()
    pltpu.sync_copy(m_sh.at[pl.ds(sid * RPT, RPT)],
                    m_out.at[cid, pl.ds(sid * RPT, RPT)])


@functools.partial(
    pl.kernel,
    out_type=jax.ShapeDtypeStruct((E,), jnp.float32),
    mesh=_mesh,
    compiler_params=_sc_params,
    scratch_types=[
        pltpu.VMEM((NCH, CH), jnp.int32),
        pltpu.VMEM((NCH, CH), jnp.int32),
        pltpu.VMEM((NP,), jnp.float32),
        pltpu.VMEM((NP,), jnp.float32),
        pltpu.VMEM((NP,), jnp.float32),
        pltpu.VMEM((NP,), jnp.float32),
        pltpu.VMEM((NP,), jnp.float32),
        pltpu.VMEM((16,), jnp.float32),
        pltpu.VMEM((EPWP,), jnp.float32),
        pltpu.SemaphoreType.DMA,
    ],
)
def _sc_line2(a, b, dinv, mparts, b3, srcw, dstw, out,
              si_v, di_v, a_v, b_v, w_v, m_v, m2_v, b3_v, o_v, sem):
    cid = lax.axis_index("c")
    sid = lax.axis_index("s")
    w = cid * NS + sid
    cps = [pltpu.async_copy(srcw.at[w], si_v, sem),
           pltpu.async_copy(dstw.at[w], di_v, sem),
           pltpu.async_copy(a, a_v.at[pl.ds(0, N)], sem),
           pltpu.async_copy(b, b_v.at[pl.ds(0, N)], sem),
           pltpu.async_copy(dinv, w_v.at[pl.ds(0, N)], sem),
           pltpu.async_copy(mparts.at[0], m_v, sem),
           pltpu.async_copy(mparts.at[1], m2_v, sem),
           pltpu.async_copy(b3, b3_v, sem)]
    def tail(i, carry):
        b_v[pl.ds(N + i * 16, 16)] = jnp.zeros((16,), jnp.float32)
        return carry
    lax.fori_loop(0, (NP - N) // 16, tail, 0)
    for cp in cps:
        cp.wait()

    def madd(i, carry):
        off = pl.ds(i * 16, 16)
        m_v[off] = m_v[off] + m2_v[off]
        return carry
    lax.fori_loop(0, NP // 16, madd, 0)
    b3v = b3_v[...]

    def chunk(k, carry):
        for j in range(CH // 16):
            s = si_v[k, pl.ds(j * 16, 16)]
            dd = di_v[k, pl.ds(j * 16, 16)]
            av = plsc.load_gather(a_v, [s])
            bv = plsc.load_gather(b_v, [dd])
            wv = plsc.load_gather(w_v, [s])
            mv = plsc.load_gather(m_v, [s])
            z = wv * mv + wv * wv * (av + bv) + b3v
            o_v[pl.ds(k * CH + j * 16, 16)] = 1.0 / (1.0 + jnp.exp(-z))
        return carry
    lax.fori_loop(0, NCH, chunk, 0)
    pltpu.sync_copy(o_v.at[pl.ds(0, EPW)], out.at[pl.ds(w * EPW, EPW)])


def _tc_prep1_body(x_ref, w1_ref, cnt_ref, g1_ref, dinv_ref):
    cnt = cnt_ref[0] + cnt_ref[1]
    dinv = lax.rsqrt(cnt + 1.0)
    h = jnp.dot(x_ref[...], w1_ref[...], preferred_element_type=jnp.float32)
    g1_ref[...] = dinv * h
    dinv_ref[...] = dinv


def _tc_prep1(x, W1, cnt31):
    return pl.pallas_call(
        _tc_prep1_body,
        grid=(GRID,),
        in_specs=[
            pl.BlockSpec((BLK, D), lambda j: (j, 0)),
            pl.BlockSpec((D, D), lambda j: (0, 0)),
            pl.BlockSpec((NC, BLK, 1), lambda j: (0, j, 0)),
        ],
        out_specs=[
            pl.BlockSpec((BLK, D), lambda j: (j, 0)),
            pl.BlockSpec((BLK, 1), lambda j: (j, 0)),
        ],
        out_shape=[
            jax.ShapeDtypeStruct((N, D), jnp.float32),
            jax.ShapeDtypeStruct((N, 1), jnp.float32),
        ],
    )(x, W1, cnt31)


def _tc_mid_body(acc_ref, g1_ref, dinv_ref, b1_ref, w2_ref, gam_ref,
                 bet_ref, g2_ref, o1_s, st_s):
    p = pl.program_id(0)
    j = pl.program_id(1)

    @pl.when(p == 0)
    def _():
        accs = acc_ref[0] + acc_ref[1] + g1_ref[...]
        o = dinv_ref[...] * accs + b1_ref[...]
        o1_s[pl.ds(j * BLK, BLK), :] = o
        s1 = jnp.sum(o, axis=0, keepdims=True)
        s2 = jnp.sum(o * o, axis=0, keepdims=True)
        st = jnp.concatenate([s1, s2], axis=0)

        @pl.when(j == 0)
        def _():
            st_s[...] = st

        @pl.when(j > 0)
        def _():
            st_s[...] = st_s[...] + st

    @pl.when(p == 1)
    def _():
        mean = st_s[0:1] / N
        var = st_s[1:2] / N - mean * mean
        inv = lax.rsqrt(var + BN_EPS)
        o1 = o1_s[pl.ds(j * BLK, BLK), :]
        o = gam_ref[...] * (o1 - mean) * inv + bet_ref[...]
        o = jnp.maximum(o, 0.0)
        h2 = jnp.dot(o, w2_ref[...], preferred_element_type=jnp.float32)
        g2_ref[...] = dinv_ref[...] * h2


def _tc_mid(acc1, g1, dinv, b1r, W2, gamr, betr):
    return pl.pallas_call(
        _tc_mid_body,
        grid=(2, GRID),
        in_specs=[
            pl.BlockSpec((NC, BLK, D), lambda p, j: (0, j * (1 - p), 0)),
            pl.BlockSpec((BLK, D), lambda p, j: (j * (1 - p), 0)),
            pl.BlockSpec((BLK, 1), lambda p, j: (j, 0)),
            pl.BlockSpec((1, D), lambda p, j: (0, 0)),
            pl.BlockSpec((D, D), lambda p, j: (0, 0)),
            pl.BlockSpec((1, D), lambda p, j: (0, 0)),
            pl.BlockSpec((1, D), lambda p, j: (0, 0)),
        ],
        out_specs=pl.BlockSpec((BLK, D), lambda p, j: (j * p, 0)),
        out_shape=jax.ShapeDtypeStruct((N, D), jnp.float32),
        scratch_shapes=[
            pltpu.VMEM((N, D), jnp.float32),
            pltpu.VMEM((2, D), jnp.float32),
        ],
    )(acc1, g1, dinv, b1r, W2, gamr, betr)


def _tc_fin_body(acc_ref, g2_ref, dinv_ref, b2_ref, w3t_ref, a_ref, b_ref):
    accs = acc_ref[0] + acc_ref[1] + g2_ref[...]
    o2 = jnp.maximum(dinv_ref[...] * accs + b2_ref[...], 0.0)
    ab = jnp.dot(o2, w3t_ref[...].T, preferred_element_type=jnp.float32)
    a_ref[...] = ab[:, 0:1]
    b_ref[...] = ab[:, 1:2]


def _tc_fin(acc2, g2, dinv, b2r, w3t):
    return pl.pallas_call(
        _tc_fin_body,
        grid=(GRID,),
        in_specs=[
            pl.BlockSpec((NC, BLK, D), lambda j: (0, j, 0)),
            pl.BlockSpec((BLK, D), lambda j: (j, 0)),
            pl.BlockSpec((BLK, 1), lambda j: (j, 0)),
            pl.BlockSpec((1, D), lambda j: (0, 0)),
            pl.BlockSpec((2, D), lambda j: (0, 0)),
        ],
        out_specs=[
            pl.BlockSpec((BLK, 1), lambda j: (j, 0)),
            pl.BlockSpec((BLK, 1), lambda j: (j, 0)),
        ],
        out_shape=[
            jax.ShapeDtypeStruct((N, 1), jnp.float32),
            jax.ShapeDtypeStruct((N, 1), jnp.float32),
        ],
    )(acc2, g2, dinv, b2r, w3t)


def kernel(x, edge_index, W1, b1, gamma1, beta1, W2, b2, W3, b3):
    src = edge_index[0].astype(jnp.int32)
    dst = edge_index[1].astype(jnp.int32)
    npad = EPWP - EPW
    padv = jnp.arange(npad, dtype=jnp.int32) % (NP - N)
    pads = jnp.broadcast_to(padv, (NW, npad))
    padd = jnp.broadcast_to(N + padv, (NW, npad))
    srcw = jnp.concatenate([src.reshape(NW, EPW), pads], axis=1)
    dstw = jnp.concatenate([dst.reshape(NW, EPW), padd], axis=1)
    srcw = srcw.reshape(NW, NCH, CH)
    dstw = dstw.reshape(NW, NCH, CH)

    b1r = b1.reshape(1, D)
    b2r = b2.reshape(1, D)
    gamr = gamma1.reshape(1, D)
    betr = beta1.reshape(1, D)
    w3t = W3.reshape(2, D)
    b3r = jnp.full((16,), b3[0], jnp.float32)

    cnt31 = _sc_count(dstw).reshape(NC, NP, 1)
    g1, dinv = _tc_prep1(x, W1, cnt31)
    acc1 = _sc_segsum(g1, srcw, dstw)
    g2 = _tc_mid(acc1, g1, dinv, b1r, W2, gamr, betr)
    acc2 = _sc_segsum(g2, srcw, dstw)
    av, bv = _tc_fin(acc2, g2, dinv, b2r, w3t)
    af = av.reshape(N)
    bf = bv.reshape(N)
    dinvf = dinv.reshape(N)
    mparts = _sc_line1(af, bf, dinvf, srcw, dstw)
    oute = _sc_line2(af, bf, dinvf, mparts, b3r, srcw, dstw)
    return oute.reshape(E, 1)
